# Initial kernel scaffold; baseline (speedup 1.0000x reference)
import jax
import jax.numpy as jnp
from jax.experimental import pallas as pl


def kernel(x, W_enc_w, b_enc, W_dec_w, b_dec):
    raise NotImplementedError("write your pallas kernel here")


# speedup vs baseline: 16.3581x; 16.3581x over previous
import functools

import jax
import jax.numpy as jnp
from jax import lax
from jax.experimental import pallas as pl
from jax.experimental.pallas import tpu as pltpu

_K = 64
_NSUB = 16
_INTERPRET = False


def _enc_body(x_ref, w_ref, be_ref, bd_ref, o_ref):
    xc = (x_ref[...] - bd_ref[...]).astype(jnp.bfloat16)
    acts = lax.dot_general(
        xc, w_ref[...].astype(jnp.bfloat16), (((1,), (1,)), ((), ())),
        preferred_element_type=jnp.float32)
    o_ref[...] = jnp.maximum(acts + be_ref[...], 0.0)


def _count_ge(arrs, t):
    c = None
    for a in arrs:
        ci = jnp.sum((a >= t).astype(jnp.int32), axis=1, keepdims=True)
        c = ci if c is None else c + ci
    return c


def _bit_search(arrs, rows, k):
    def body(it, lo):
        t = jnp.bitwise_or(lo, jnp.left_shift(jnp.int32(1), 30 - it))
        return jnp.where(_count_ge(arrs, t) >= k, t, lo)
    return lax.fori_loop(0, 31, body, jnp.zeros((rows, 1), jnp.int32))


def _sel_body(p_ref, h_ref, l0_ref, *, n_sub, k):
    i = pl.program_id(0)
    p = p_ref[...]
    bi, f = p.shape
    seg = f // n_sub
    neg = jnp.full((bi, seg), -1.0, jnp.float32)
    m1 = m2 = m3 = m4 = neg
    for s in range(n_sub):
        v = p[:, s * seg:(s + 1) * seg]
        h1 = jnp.maximum(m1, v); v = jnp.minimum(m1, v); m1 = h1
        h2 = jnp.maximum(m2, v); v = jnp.minimum(m2, v); m2 = h2
        h3 = jnp.maximum(m3, v); v = jnp.minimum(m3, v); m3 = h3
        m4 = jnp.maximum(m4, v)
    s_i32 = [lax.bitcast_convert_type(m, jnp.int32) for m in (m1, m2, m3, m4)]
    p_i32 = lax.bitcast_convert_type(p, jnp.int32)

    t_s = _bit_search(s_i32, bi, k)
    c_star = _count_ge([p_i32], t_s)
    t_final = lax.cond(
        jnp.any(c_star != k),
        lambda: _bit_search([p_i32], bi, k),
        lambda: t_s)
    hv = jnp.where(p_i32 >= t_final, p, 0.0)
    h_ref[...] = hv

    @pl.when(i == 0)
    def _():
        l0_ref[...] = jnp.zeros_like(l0_ref)
    l0_ref[...] += jnp.sum((hv > 0).astype(jnp.float32))[None, None]


def _dec_body(h_ref, wd_ref, x_ref, bd_ref,
              xhat_ref, srr_ref, sx_ref, sxx_ref, *, nj):
    i = pl.program_id(0)
    j = pl.program_id(1)
    hb = h_ref[...].astype(jnp.bfloat16)
    part = lax.dot_general(
        hb, wd_ref[...], (((1,), (1,)), ((), ())),
        preferred_element_type=jnp.float32)

    @pl.when(j == 0)
    def _():
        xhat_ref[...] = part

    @pl.when(j > 0)
    def _():
        xhat_ref[...] += part

    @pl.when((i == 0) & (j == 0))
    def _():
        srr_ref[...] = jnp.zeros_like(srr_ref)
        sx_ref[...] = jnp.zeros_like(sx_ref)
        sxx_ref[...] = jnp.zeros_like(sxx_ref)

    @pl.when(j == nj - 1)
    def _():
        xhat = xhat_ref[...] + bd_ref[...]
        xhat_ref[...] = xhat
        xv = x_ref[...]
        rd = xhat - xv
        srr_ref[...] += jnp.sum(rd * rd)[None, None]
        sx_ref[...] += jnp.sum(xv, axis=0, keepdims=True)
        sxx_ref[...] += jnp.sum(xv * xv)[None, None]


def kernel(x, W_enc_w, b_enc, W_dec_w, b_dec):
    n, d = x.shape
    f = W_enc_w.shape[0]
    k = _K
    bi1, bj1 = min(512, n), min(2048, f)
    bi2 = min(64, n)
    bi3, bj3 = min(1024, n), min(2048, f)

    w_dec_bf = W_dec_w.astype(jnp.bfloat16)
    be2 = b_enc.reshape(1, f)
    bd2 = b_dec.reshape(1, d)

    pre = pl.pallas_call(
        _enc_body,
        grid=(f // bj1, n // bi1),
        in_specs=[
            pl.BlockSpec((bi1, d), lambda j, i: (i, 0)),
            pl.BlockSpec((bj1, d), lambda j, i: (j, 0)),
            pl.BlockSpec((1, bj1), lambda j, i: (0, j)),
            pl.BlockSpec((1, d), lambda j, i: (0, 0)),
        ],
        out_specs=pl.BlockSpec((bi1, bj1), lambda j, i: (i, j)),
        out_shape=jax.ShapeDtypeStruct((n, f), jnp.float32),
        compiler_params=pltpu.CompilerParams(
            dimension_semantics=("arbitrary", "arbitrary")),
        interpret=_INTERPRET,
    )(x, W_enc_w, be2, bd2)

    h, l0s = pl.pallas_call(
        functools.partial(_sel_body, n_sub=_NSUB, k=k),
        grid=(n // bi2,),
        in_specs=[pl.BlockSpec((bi2, f), lambda i: (i, 0))],
        out_specs=[
            pl.BlockSpec((bi2, f), lambda i: (i, 0)),
            pl.BlockSpec((1, 1), lambda i: (0, 0)),
        ],
        out_shape=[
            jax.ShapeDtypeStruct((n, f), jnp.float32),
            jax.ShapeDtypeStruct((1, 1), jnp.float32),
        ],
        compiler_params=pltpu.CompilerParams(
            dimension_semantics=("arbitrary",)),
        interpret=_INTERPRET,
    )(pre)

    xhat, srr, sx, sxx = pl.pallas_call(
        functools.partial(_dec_body, nj=f // bj3),
        grid=(n // bi3, f // bj3),
        in_specs=[
            pl.BlockSpec((bi3, bj3), lambda i, j: (i, j)),
            pl.BlockSpec((d, bj3), lambda i, j: (0, j)),
            pl.BlockSpec((bi3, d), lambda i, j: (i, 0)),
            pl.BlockSpec((1, d), lambda i, j: (0, 0)),
        ],
        out_specs=[
            pl.BlockSpec((bi3, d), lambda i, j: (i, 0)),
            pl.BlockSpec((1, 1), lambda i, j: (0, 0)),
            pl.BlockSpec((1, d), lambda i, j: (0, 0)),
            pl.BlockSpec((1, 1), lambda i, j: (0, 0)),
        ],
        out_shape=[
            jax.ShapeDtypeStruct((n, d), jnp.float32),
            jax.ShapeDtypeStruct((1, 1), jnp.float32),
            jax.ShapeDtypeStruct((1, d), jnp.float32),
            jax.ShapeDtypeStruct((1, 1), jnp.float32),
        ],
        compiler_params=pltpu.CompilerParams(
            dimension_semantics=("arbitrary", "arbitrary")),
        interpret=_INTERPRET,
    )(h, w_dec_bf, x, bd2)

    recon_loss = srr[0, 0] / (n * d)
    l0 = l0s[0, 0] / n
    total_var = sxx[0, 0] - jnp.sum(sx * sx) / n
    explained_var = 1.0 - srr[0, 0] / (total_var + 1e-8)
    return (xhat, h, recon_loss, l0, explained_var)
